# Initial kernel scaffold; baseline (speedup 1.0000x reference)
import jax
import jax.numpy as jnp
from jax.experimental import pallas as pl


def kernel(x, edge_index, edge_attr, batch, W_node, b_node, W_edge, b_edge, W1_0, b1_0, W2_0, b2_0, g_0, be_0, W1_1, b1_1, W2_1, b2_1, g_1, be_1, W_lin1, b_lin1, W_lin2, b_lin2):
    raise NotImplementedError("write your pallas kernel here")


# speedup vs baseline: 2.6920x; 2.6920x over previous
import functools

import jax
import jax.numpy as jnp
from jax import lax
from jax.experimental import pallas as pl
from jax.experimental.pallas import tpu as pltpu
from jax.experimental.pallas import tpu_sc as plsc

G = 64
BN_EPS = 1e-5
CHUNK = 128
NCORES = 2
NSUB = 16
NTILES = NCORES * NSUB
NEG = -1e30


def _embed_body(x_ref, w_ref, b_ref, o_ref):
    o_ref[...] = (
        jnp.dot(x_ref[...], w_ref[...], preferred_element_type=jnp.float32)
        + b_ref[...]
    )


def _edge_embed_body(ea_ref, w_ref, b_ref, o_ref, *, blk, n_real):
    pid = pl.program_id(0)
    y = (
        jnp.dot(ea_ref[...], w_ref[...], preferred_element_type=jnp.float32)
        + b_ref[...]
    )
    row = pid * blk + lax.broadcasted_iota(jnp.int32, (blk, 1), 0)
    o_ref[...] = jnp.where(row < n_real, y, NEG)


def _mlp_stats_body(h_ref, a_ref, w1_ref, b1_ref, w2_ref, b2_ref,
                    z_ref, s_ref, acc, *, nblk):
    i = pl.program_id(0)
    u = h_ref[...] + a_ref[0] + a_ref[1]
    t = jnp.maximum(
        jnp.dot(u, w1_ref[...], preferred_element_type=jnp.float32)
        + b1_ref[...], 0.0)
    z = (jnp.dot(t, w2_ref[...], preferred_element_type=jnp.float32)
         + b2_ref[...])
    z_ref[...] = z

    @pl.when(i == 0)
    def _():
        acc[...] = jnp.zeros_like(acc)

    acc[...] += jnp.concatenate(
        [jnp.sum(z, axis=0, keepdims=True),
         jnp.sum(z * z, axis=0, keepdims=True)], axis=0)

    @pl.when(i == nblk - 1)
    def _():
        s_ref[...] = acc[...]


def _bn_body(z_ref, s_ref, g_ref, be_ref, o_ref, *, n):
    mu = s_ref[0:1, :] / n
    var = s_ref[1:2, :] / n - mu * mu
    inv = lax.rsqrt(var + BN_EPS)
    o_ref[...] = jnp.maximum(
        (z_ref[...] - mu) * inv * g_ref[...] + be_ref[...], 0.0)


def _pool_head_body(z_ref, s_ref, g_ref, be_ref, bf_ref,
                    wl1_ref, bl1_ref, wl2_ref, bl2_ref,
                    o_ref, accs, accc, *, n, nblk, blk):
    i = pl.program_id(0)
    mu = s_ref[0:1, :] / n
    var = s_ref[1:2, :] / n - mu * mu
    inv = lax.rsqrt(var + BN_EPS)
    h = jnp.maximum(
        (z_ref[...] - mu) * inv * g_ref[...] + be_ref[...], 0.0)
    gid = bf_ref[...]
    iota = lax.broadcasted_iota(jnp.int32, (blk, G), 1)
    onehot = jnp.where(gid == iota, 1.0, 0.0)
    gs = lax.dot_general(onehot, h, (((0,), (0,)), ((), ())),
                         preferred_element_type=jnp.float32)
    cnt = lax.dot_general(onehot, jnp.ones((blk, 1), jnp.float32),
                          (((0,), (0,)), ((), ())),
                          preferred_element_type=jnp.float32)

    @pl.when(i == 0)
    def _():
        accs[...] = jnp.zeros_like(accs)
        accc[...] = jnp.zeros_like(accc)

    accs[...] += gs
    accc[...] += cnt

    @pl.when(i == nblk - 1)
    def _():
        gx = accs[...] / jnp.maximum(accc[...], 1.0)
        t = jnp.maximum(
            jnp.dot(gx, wl1_ref[...], preferred_element_type=jnp.float32)
            + bl1_ref[...], 0.0)
        o_ref[...] = (
            jnp.dot(t, wl2_ref[...], preferred_element_type=jnp.float32)
            + bl2_ref[...])


def _full(shape):
    return pl.BlockSpec(shape, lambda *_: tuple(0 for _ in shape))


def _make_sc_agg(n, h_dim, cpt, e_pad):
    mesh = plsc.VectorSubcoreMesh(core_axis_name="c", subcore_axis_name="s")
    rows_per_tile = -(-n // (NSUB * CHUNK)) * CHUNK
    n_pad = NSUB * rows_per_tile
    nq = h_dim // 16

    @functools.partial(
        pl.kernel,
        out_type=jax.ShapeDtypeStruct((NCORES, n_pad, h_dim), jnp.float32),
        mesh=mesh,
        scratch_types=[
            pltpu.VMEM((cpt, CHUNK), jnp.int32),
            pltpu.VMEM((cpt, CHUNK), jnp.int32),
            pltpu.VMEM((CHUNK, h_dim), jnp.float32),
            pltpu.VMEM((CHUNK, h_dim), jnp.float32),
            pltpu.VMEM_SHARED((n_pad, h_dim), jnp.float32),
            pltpu.SemaphoreType.DMA,
            pltpu.SemaphoreType.DMA,
        ],
        compiler_params=pltpu.CompilerParams(use_tc_tiling_on_sc=False),
    )
    def sc_agg(h_hbm, e_hbm, src_hbm, dst_hbm, out_hbm,
               src_v, dst_v, hs_v, e_v, agg_sh, sem_g, sem_e):
        c = lax.axis_index("c")
        s = lax.axis_index("s")
        wid = c * NSUB + s
        pltpu.sync_copy(src_hbm.at[wid], src_v)
        pltpu.sync_copy(dst_hbm.at[wid], dst_v)

        def zrow(i, carry):
            for q in range(nq):
                hs_v[i, pl.ds(q * 16, 16)] = jnp.zeros((16,), jnp.float32)
            return carry
        lax.fori_loop(0, CHUNK, zrow, 0)

        def zcp(k, carry):
            pltpu.sync_copy(
                hs_v,
                agg_sh.at[pl.ds(s * rows_per_tile + k * CHUNK, CHUNK)])
            return carry
        lax.fori_loop(0, rows_per_tile // CHUNK, zcp, 0)
        plsc.subcore_barrier()

        def chunk_body(j, carry):
            cg = pltpu.async_copy(h_hbm.at[src_v.at[j]], hs_v, sem_g)
            ce = pltpu.async_copy(
                e_hbm.at[pl.ds((wid * cpt + j) * CHUNK, CHUNK)], e_v, sem_e)
            cg.wait()
            ce.wait()

            def crow(i, cc):
                for q in range(nq):
                    sl = pl.ds(q * 16, 16)
                    hs_v[i, sl] = jnp.maximum(hs_v[i, sl] + e_v[i, sl], 0.0)
                return cc
            lax.fori_loop(0, CHUNK, crow, 0)
            pltpu.sync_copy(hs_v, agg_sh.at[dst_v.at[j]], add=True)
            return carry
        lax.fori_loop(0, cpt, chunk_body, 0)

        plsc.subcore_barrier()
        pltpu.sync_copy(
            agg_sh.at[pl.ds(s * rows_per_tile, rows_per_tile)],
            out_hbm.at[c, pl.ds(s * rows_per_tile, rows_per_tile)])

    return sc_agg


def kernel(x, edge_index, edge_attr, batch,
           W_node, b_node, W_edge, b_edge,
           W1_0, b1_0, W2_0, b2_0, g_0, be_0,
           W1_1, b1_1, W2_1, b2_1, g_1, be_1,
           W_lin1, b_lin1, W_lin2, b_lin2):
    n, df = x.shape
    e_cnt, de = edge_attr.shape
    h_dim = W_node.shape[1]
    f32 = jnp.float32

    ei = edge_index.astype(jnp.int32)
    cpt = -(-e_cnt // (NTILES * CHUNK))
    e_pad = NTILES * cpt * CHUNK
    pad = e_pad - e_cnt
    zi = jnp.zeros((pad,), jnp.int32)
    src_p = jnp.concatenate([ei[0], zi]).reshape(NTILES, cpt, CHUNK)
    dst_p = jnp.concatenate([ei[1], zi]).reshape(NTILES, cpt, CHUNK)
    ea_p = jnp.concatenate(
        [edge_attr, jnp.zeros((pad, de), f32)], axis=0)
    batch_f = batch.astype(jnp.int32).reshape(n, 1)
    b_node2 = b_node.reshape(1, -1)
    b_edge2 = b_edge.reshape(1, -1)
    b1_0_2, b2_0_2 = b1_0.reshape(1, -1), b2_0.reshape(1, -1)
    b1_1_2, b2_1_2 = b1_1.reshape(1, -1), b2_1.reshape(1, -1)
    g_0_2, be_0_2 = g_0.reshape(1, -1), be_0.reshape(1, -1)
    g_1_2, be_1_2 = g_1.reshape(1, -1), be_1.reshape(1, -1)
    bl1_2, bl2_2 = b_lin1.reshape(1, -1), b_lin2.reshape(1, -1)

    h0 = pl.pallas_call(
        _embed_body,
        out_shape=jax.ShapeDtypeStruct((n, h_dim), f32),
    )(x, W_node, b_node2)

    eblk = 512
    egrid = e_pad // eblk
    e_emb = pl.pallas_call(
        functools.partial(_edge_embed_body, blk=eblk, n_real=e_cnt),
        grid=(egrid,),
        in_specs=[
            pl.BlockSpec((eblk, de), lambda i: (i, 0)),
            _full((de, h_dim)),
            _full((1, h_dim)),
        ],
        out_specs=pl.BlockSpec((eblk, h_dim), lambda i: (i, 0)),
        out_shape=jax.ShapeDtypeStruct((e_pad, h_dim), f32),
    )(ea_p, W_edge, b_edge2)

    sc_agg = _make_sc_agg(n, h_dim, cpt, e_pad)

    nblk = 1000
    ngrid = n // nblk

    def mlp_stats(h, agg2, w1, b1r, w2, b2r):
        d2 = w1.shape[1]
        return pl.pallas_call(
            functools.partial(_mlp_stats_body, nblk=ngrid),
            grid=(ngrid,),
            in_specs=[
                pl.BlockSpec((nblk, h_dim), lambda i: (i, 0)),
                pl.BlockSpec((NCORES, nblk, h_dim), lambda i: (0, i, 0)),
                _full((h_dim, d2)),
                _full((1, d2)),
                _full((d2, h_dim)),
                _full((1, h_dim)),
            ],
            out_specs=[
                pl.BlockSpec((nblk, h_dim), lambda i: (i, 0)),
                _full((2, h_dim)),
            ],
            out_shape=[
                jax.ShapeDtypeStruct((n, h_dim), f32),
                jax.ShapeDtypeStruct((2, h_dim), f32),
            ],
            scratch_shapes=[pltpu.VMEM((2, h_dim), f32)],
        )(h, agg2, w1, b1r, w2, b2r)

    def bn_apply(z, stats, gr, ber):
        return pl.pallas_call(
            functools.partial(_bn_body, n=float(n)),
            grid=(ngrid,),
            in_specs=[
                pl.BlockSpec((nblk, h_dim), lambda i: (i, 0)),
                _full((2, h_dim)),
                _full((1, h_dim)),
                _full((1, h_dim)),
            ],
            out_specs=pl.BlockSpec((nblk, h_dim), lambda i: (i, 0)),
            out_shape=jax.ShapeDtypeStruct((n, h_dim), f32),
        )(z, stats, gr, ber)

    agg0 = sc_agg(h0, e_emb, src_p, dst_p)
    z0, s0 = mlp_stats(h0, agg0, W1_0, b1_0_2, W2_0, b2_0_2)
    h1 = bn_apply(z0, s0, g_0_2, be_0_2)

    agg1 = sc_agg(h1, e_emb, src_p, dst_p)
    z1, s1 = mlp_stats(h1, agg1, W1_1, b1_1_2, W2_1, b2_1_2)

    out = pl.pallas_call(
        functools.partial(_pool_head_body, n=float(n), nblk=ngrid, blk=nblk),
        grid=(ngrid,),
        in_specs=[
            pl.BlockSpec((nblk, h_dim), lambda i: (i, 0)),
            _full((2, h_dim)),
            _full((1, h_dim)),
            _full((1, h_dim)),
            pl.BlockSpec((nblk, 1), lambda i: (i, 0)),
            _full((h_dim, W_lin1.shape[1])),
            _full((1, W_lin1.shape[1])),
            _full((W_lin1.shape[1], 1)),
            _full((1, 1)),
        ],
        out_specs=_full((G, 1)),
        out_shape=jax.ShapeDtypeStruct((G, 1), f32),
        scratch_shapes=[
            pltpu.VMEM((G, h_dim), f32),
            pltpu.VMEM((G, 1), f32),
        ],
    )(z1, s1, g_1_2, be_1_2, batch_f,
      W_lin1, bl1_2, W_lin2, bl2_2)
    return out
